# baseline (speedup 1.0000x reference)
import jax
import jax.numpy as jnp
from jax.experimental import pallas as pl
from jax.experimental.pallas import tpu as pltpu


def _rows_kernel(x_ref, w_ref, wt_ref, b_ref, o_ref):
    x = x_ref[...].astype(jnp.bfloat16)
    h = jnp.dot(x, w_ref[...], preferred_element_type=jnp.float32)
    y = jnp.dot(h.astype(jnp.bfloat16), wt_ref[...],
                preferred_element_type=jnp.float32)
    o_ref[...] = jnp.maximum(y + b_ref[...], 0.0).astype(o_ref.dtype)


def _round_up(v, m):
    return ((v + m - 1) // m) * m


def kernel(x, w1, b):
    n_data, n_feat = x.shape
    nf_w, n_hidden = w1.shape
    assert nf_w == n_feat

    b2d = b.reshape(1, n_feat)
    w_bf = w1.astype(jnp.bfloat16)
    wt_bf = w_bf.T

    tm = min(2048, _round_up(n_data, 8))
    grid = (pl.cdiv(n_data, tm),)

    main_cost = pl.CostEstimate(
        flops=4 * n_data * n_feat * n_hidden,
        transcendentals=0,
        bytes_accessed=2 * n_data * n_feat * 4 + 2 * n_feat * n_hidden * 2,
    )
    return pl.pallas_call(
        _rows_kernel,
        out_shape=jax.ShapeDtypeStruct((n_data, n_feat), x.dtype),
        grid=grid,
        in_specs=[
            pl.BlockSpec((tm, n_feat), lambda i: (i, 0)),
            pl.BlockSpec((n_feat, n_hidden), lambda i: (0, 0)),
            pl.BlockSpec((n_hidden, n_feat), lambda i: (0, 0)),
            pl.BlockSpec((1, n_feat), lambda i: (0, 0)),
        ],
        out_specs=pl.BlockSpec((tm, n_feat), lambda i: (i, 0)),
        cost_estimate=main_cost,
        compiler_params=pltpu.CompilerParams(
            dimension_semantics=("parallel",),
            vmem_limit_bytes=48 * 1024 * 1024,
        ),
    )(x, w_bf, wt_bf, b2d)


# speedup vs baseline: 1.4719x; 1.4719x over previous
import jax
import jax.numpy as jnp
from jax.experimental import pallas as pl
from jax.experimental.pallas import tpu as pltpu


def _gram_kernel(w_ref, g_ref):
    w = w_ref[...].astype(jnp.bfloat16)
    g = jax.lax.dot_general(
        w, w, (((1,), (1,)), ((), ())),
        preferred_element_type=jnp.float32,
    )
    g_ref[...] = g.astype(jnp.bfloat16)


def _rows_kernel(x_ref, g_ref, b_ref, o_ref):
    x = x_ref[...].astype(jnp.bfloat16)
    y = jnp.dot(x, g_ref[...], preferred_element_type=jnp.float32)
    o_ref[...] = jnp.maximum(y + b_ref[...], 0.0).astype(o_ref.dtype)


def _round_up(v, m):
    return ((v + m - 1) // m) * m


def kernel(x, w1, b):
    n_data, n_feat = x.shape
    nf_w, n_hidden = w1.shape
    assert nf_w == n_feat

    b2d = b.reshape(1, n_feat)

    gram_cost = pl.CostEstimate(
        flops=2 * n_feat * n_feat * n_hidden,
        transcendentals=0,
        bytes_accessed=n_feat * n_hidden * 4 + n_feat * n_feat * 2,
    )
    g = pl.pallas_call(
        _gram_kernel,
        out_shape=jax.ShapeDtypeStruct((n_feat, n_feat), jnp.bfloat16),
        in_specs=[pl.BlockSpec(memory_space=pltpu.MemorySpace.VMEM)],
        out_specs=pl.BlockSpec(memory_space=pltpu.MemorySpace.VMEM),
        cost_estimate=gram_cost,
    )(w1)

    tm = min(2736, _round_up(n_data, 8))
    grid = (pl.cdiv(n_data, tm),)

    main_cost = pl.CostEstimate(
        flops=2 * n_data * n_feat * n_feat,
        transcendentals=0,
        bytes_accessed=2 * n_data * n_feat * 4 + n_feat * n_feat * 2,
    )
    return pl.pallas_call(
        _rows_kernel,
        out_shape=jax.ShapeDtypeStruct((n_data, n_feat), x.dtype),
        grid=grid,
        in_specs=[
            pl.BlockSpec((tm, n_feat), lambda i: (i, 0)),
            pl.BlockSpec((n_feat, n_feat), lambda i: (0, 0)),
            pl.BlockSpec((1, n_feat), lambda i: (0, 0)),
        ],
        out_specs=pl.BlockSpec((tm, n_feat), lambda i: (i, 0)),
        cost_estimate=main_cost,
        compiler_params=pltpu.CompilerParams(
            dimension_semantics=("parallel",),
            vmem_limit_bytes=58 * 1024 * 1024,
        ),
    )(x, g, b2d)


# speedup vs baseline: 1.5959x; 1.0843x over previous
import functools

import jax
import jax.numpy as jnp
from jax.experimental import pallas as pl
from jax.experimental.pallas import tpu as pltpu


def _fused_kernel(x_hbm, w_ref, b_ref, o_hbm,
                  x_buf, o_buf, g_buf, in_sem, out_sem,
                  *, block: int, n_steps: int):
    def dma_in(slot, step):
        pltpu.make_async_copy(
            x_hbm.at[pl.ds(step * block, block), :],
            x_buf.at[slot], in_sem.at[slot]).start()

    def wait_in(slot):
        pltpu.make_async_copy(
            x_hbm.at[pl.ds(0, block), :],
            x_buf.at[slot], in_sem.at[slot]).wait()

    def dma_out(slot, step):
        pltpu.make_async_copy(
            o_buf.at[slot],
            o_hbm.at[pl.ds(step * block, block), :], out_sem.at[slot]).start()

    def wait_out(slot):
        pltpu.make_async_copy(
            o_buf.at[slot],
            o_hbm.at[pl.ds(0, block), :], out_sem.at[slot]).wait()

    dma_in(0, 0)
    w = w_ref[...].astype(jnp.bfloat16)
    g_buf[...] = jax.lax.dot_general(
        w, w, (((1,), (1,)), ((), ())),
        preferred_element_type=jnp.float32).astype(jnp.bfloat16)

    def body(step, _):
        cur = jax.lax.rem(step, 2)
        nxt = jax.lax.rem(step + 1, 2)

        @pl.when(step + 1 < n_steps)
        def _():
            dma_in(nxt, step + 1)

        wait_in(cur)

        @pl.when(step >= 2)
        def _():
            wait_out(cur)

        xt = x_buf[cur].astype(jnp.bfloat16)
        y = jnp.dot(xt, g_buf[...], preferred_element_type=jnp.float32)
        o_buf[cur] = jnp.maximum(y + b_ref[...], 0.0).astype(o_buf.dtype)

        dma_out(cur, step)
        return ()

    jax.lax.fori_loop(0, n_steps, body, (), unroll=False)
    if n_steps >= 2:
        wait_out(jax.lax.rem(n_steps - 2, 2))
    wait_out(jax.lax.rem(n_steps - 1, 2))


def kernel(x, w1, b):
    n_data, n_feat = x.shape
    nf_w, n_hidden = w1.shape
    assert nf_w == n_feat

    b2d = b.reshape(1, n_feat)

    block = n_data
    for cand in (2048, 1024, 512, 256, 128, 64, 32, 16, 8):
        if n_data % cand == 0:
            block = cand
            break
    n_steps = n_data // block

    cost = pl.CostEstimate(
        flops=2 * n_data * n_feat * n_feat + 2 * n_feat * n_feat * n_hidden,
        transcendentals=0,
        bytes_accessed=2 * n_data * n_feat * 4 + n_feat * n_hidden * 4,
    )
    return pl.pallas_call(
        functools.partial(_fused_kernel, block=block, n_steps=n_steps),
        out_shape=jax.ShapeDtypeStruct((n_data, n_feat), x.dtype),
        in_specs=[
            pl.BlockSpec(memory_space=pltpu.MemorySpace.HBM),
            pl.BlockSpec(memory_space=pltpu.MemorySpace.VMEM),
            pl.BlockSpec(memory_space=pltpu.MemorySpace.VMEM),
        ],
        out_specs=pl.BlockSpec(memory_space=pltpu.MemorySpace.HBM),
        scratch_shapes=[
            pltpu.VMEM((2, block, n_feat), x.dtype),
            pltpu.VMEM((2, block, n_feat), x.dtype),
            pltpu.VMEM((n_feat, n_feat), jnp.bfloat16),
            pltpu.SemaphoreType.DMA((2,)),
            pltpu.SemaphoreType.DMA((2,)),
        ],
        cost_estimate=cost,
        compiler_params=pltpu.CompilerParams(
            vmem_limit_bytes=58 * 1024 * 1024,
        ),
    )(x, w1, b2d)


# speedup vs baseline: 1.7051x; 1.0684x over previous
import functools

import jax
import jax.numpy as jnp
from jax.experimental import pallas as pl
from jax.experimental.pallas import tpu as pltpu


def _fused_kernel(x_hbm, w_ref, b_ref, o_hbm,
                  x_buf, o_buf, g_buf, in_sem, out_sem,
                  *, block: int, n_steps: int, in_slots: int, out_slots: int):
    def dma_in(slot, step):
        pltpu.make_async_copy(
            x_hbm.at[pl.ds(step * block, block), :],
            x_buf.at[slot], in_sem.at[slot]).start()

    def wait_in(slot):
        pltpu.make_async_copy(
            x_hbm.at[pl.ds(0, block), :],
            x_buf.at[slot], in_sem.at[slot]).wait()

    def dma_out(slot, step):
        pltpu.make_async_copy(
            o_buf.at[slot],
            o_hbm.at[pl.ds(step * block, block), :], out_sem.at[slot]).start()

    def wait_out(slot):
        pltpu.make_async_copy(
            o_buf.at[slot],
            o_hbm.at[pl.ds(0, block), :], out_sem.at[slot]).wait()

    for s in range(min(in_slots - 1, n_steps)):
        dma_in(s, s)
    w = w_ref[...].astype(jnp.bfloat16)
    g_buf[...] = jax.lax.dot_general(
        w, w, (((1,), (1,)), ((), ())),
        preferred_element_type=jnp.float32).astype(jnp.bfloat16)

    def body(step, _):
        cur_in = jax.lax.rem(step, in_slots)
        cur_out = jax.lax.rem(step, out_slots)

        @pl.when(step + in_slots - 1 < n_steps)
        def _():
            dma_in(jax.lax.rem(step + in_slots - 1, in_slots),
                   step + in_slots - 1)

        wait_in(cur_in)

        @pl.when(step >= out_slots)
        def _():
            wait_out(cur_out)

        xt = x_buf[cur_in].astype(jnp.bfloat16)
        y = jnp.dot(xt, g_buf[...], preferred_element_type=jnp.float32)
        o_buf[cur_out] = jnp.maximum(y + b_ref[...], 0.0).astype(o_buf.dtype)

        dma_out(cur_out, step)
        return ()

    jax.lax.fori_loop(0, n_steps, body, (), unroll=False)
    for s in range(max(n_steps - out_slots, 0), n_steps):
        wait_out(s % out_slots)


def kernel(x, w1, b):
    n_data, n_feat = x.shape
    nf_w, n_hidden = w1.shape
    assert nf_w == n_feat

    b2d = b.reshape(1, n_feat)

    block = n_data
    for cand in (2048, 1024, 512, 256, 128, 64, 32, 16, 8):
        if n_data % cand == 0:
            block = cand
            break
    n_steps = n_data // block
    in_slots = min(3, max(n_steps, 2))
    out_slots = 2

    cost = pl.CostEstimate(
        flops=2 * n_data * n_feat * n_feat + 2 * n_feat * n_feat * n_hidden,
        transcendentals=0,
        bytes_accessed=2 * n_data * n_feat * 4 + n_feat * n_hidden * 4,
    )
    return pl.pallas_call(
        functools.partial(_fused_kernel, block=block, n_steps=n_steps,
                          in_slots=in_slots, out_slots=out_slots),
        out_shape=jax.ShapeDtypeStruct((n_data, n_feat), x.dtype),
        in_specs=[
            pl.BlockSpec(memory_space=pltpu.MemorySpace.HBM),
            pl.BlockSpec(memory_space=pltpu.MemorySpace.VMEM),
            pl.BlockSpec(memory_space=pltpu.MemorySpace.VMEM),
        ],
        out_specs=pl.BlockSpec(memory_space=pltpu.MemorySpace.HBM),
        scratch_shapes=[
            pltpu.VMEM((in_slots, block, n_feat), x.dtype),
            pltpu.VMEM((out_slots, block, n_feat), x.dtype),
            pltpu.VMEM((n_feat, n_feat), jnp.bfloat16),
            pltpu.SemaphoreType.DMA((in_slots,)),
            pltpu.SemaphoreType.DMA((out_slots,)),
        ],
        cost_estimate=cost,
        compiler_params=pltpu.CompilerParams(
            vmem_limit_bytes=58 * 1024 * 1024,
        ),
    )(x, w1, b2d)


# speedup vs baseline: 1.7200x; 1.0088x over previous
import functools

import jax
import jax.numpy as jnp
from jax.experimental import pallas as pl
from jax.experimental.pallas import tpu as pltpu


def _fused_kernel(x_hbm, w_ref, b_ref, o_hbm,
                  x_buf, o_buf, g_buf, in_sem, out_sem,
                  *, block: int, n_steps: int, in_slots: int, out_slots: int):
    half = block // 2

    def dma_in(slot, step):
        base = step * block
        pltpu.make_async_copy(
            x_hbm.at[pl.ds(base, half), :],
            x_buf.at[slot, pl.ds(0, half), :], in_sem.at[slot, 0]).start()
        pltpu.make_async_copy(
            x_hbm.at[pl.ds(base + half, half), :],
            x_buf.at[slot, pl.ds(half, half), :], in_sem.at[slot, 1]).start()

    def wait_in(slot):
        for s in range(2):
            pltpu.make_async_copy(
                x_hbm.at[pl.ds(0, half), :],
                x_buf.at[slot, pl.ds(0, half), :], in_sem.at[slot, s]).wait()

    def dma_out(slot, step):
        base = step * block
        pltpu.make_async_copy(
            o_buf.at[slot, pl.ds(0, half), :],
            o_hbm.at[pl.ds(base, half), :], out_sem.at[slot, 0]).start()
        pltpu.make_async_copy(
            o_buf.at[slot, pl.ds(half, half), :],
            o_hbm.at[pl.ds(base + half, half), :], out_sem.at[slot, 1]).start()

    def wait_out(slot):
        for s in range(2):
            pltpu.make_async_copy(
                o_buf.at[slot, pl.ds(0, half), :],
                o_hbm.at[pl.ds(0, half), :], out_sem.at[slot, s]).wait()

    for s in range(min(in_slots - 1, n_steps)):
        dma_in(s, s)
    w = w_ref[...].astype(jnp.bfloat16)
    g_buf[...] = jax.lax.dot_general(
        w, w, (((1,), (1,)), ((), ())),
        preferred_element_type=jnp.float32).astype(jnp.bfloat16)

    def body(step, _):
        cur_in = jax.lax.rem(step, in_slots)
        cur_out = jax.lax.rem(step, out_slots)

        @pl.when(step + in_slots - 1 < n_steps)
        def _():
            dma_in(jax.lax.rem(step + in_slots - 1, in_slots),
                   step + in_slots - 1)

        wait_in(cur_in)

        @pl.when(step >= out_slots)
        def _():
            wait_out(cur_out)

        xt = x_buf[cur_in].astype(jnp.bfloat16)
        y = jnp.dot(xt, g_buf[...], preferred_element_type=jnp.float32)
        o_buf[cur_out] = jnp.maximum(y + b_ref[...], 0.0).astype(o_buf.dtype)

        dma_out(cur_out, step)
        return ()

    jax.lax.fori_loop(0, n_steps, body, (), unroll=False)
    for s in range(max(n_steps - out_slots, 0), n_steps):
        wait_out(s % out_slots)


def kernel(x, w1, b):
    n_data, n_feat = x.shape
    nf_w, n_hidden = w1.shape
    assert nf_w == n_feat

    b2d = b.reshape(1, n_feat)

    block = n_data
    for cand in (2048, 1024, 512, 256, 128, 64, 32, 16):
        if n_data % cand == 0:
            block = cand
            break
    n_steps = n_data // block
    in_slots = min(3, max(n_steps, 2))
    out_slots = 2

    cost = pl.CostEstimate(
        flops=2 * n_data * n_feat * n_feat + 2 * n_feat * n_feat * n_hidden,
        transcendentals=0,
        bytes_accessed=2 * n_data * n_feat * 4 + n_feat * n_hidden * 4,
    )
    return pl.pallas_call(
        functools.partial(_fused_kernel, block=block, n_steps=n_steps,
                          in_slots=in_slots, out_slots=out_slots),
        out_shape=jax.ShapeDtypeStruct((n_data, n_feat), x.dtype),
        in_specs=[
            pl.BlockSpec(memory_space=pltpu.MemorySpace.HBM),
            pl.BlockSpec(memory_space=pltpu.MemorySpace.VMEM),
            pl.BlockSpec(memory_space=pltpu.MemorySpace.VMEM),
        ],
        out_specs=pl.BlockSpec(memory_space=pltpu.MemorySpace.HBM),
        scratch_shapes=[
            pltpu.VMEM((in_slots, block, n_feat), x.dtype),
            pltpu.VMEM((out_slots, block, n_feat), x.dtype),
            pltpu.VMEM((n_feat, n_feat), jnp.bfloat16),
            pltpu.SemaphoreType.DMA((in_slots, 2)),
            pltpu.SemaphoreType.DMA((out_slots, 2)),
        ],
        cost_estimate=cost,
        compiler_params=pltpu.CompilerParams(
            vmem_limit_bytes=58 * 1024 * 1024,
        ),
    )(x, w1, b2d)


# speedup vs baseline: 1.8393x; 1.0693x over previous
import functools

import jax
import jax.numpy as jnp
from jax.experimental import pallas as pl
from jax.experimental.pallas import tpu as pltpu


def _fused_kernel(x_hbm, w_ref, b_ref, o_hbm,
                  x_buf, o_buf, g_buf, in_sem, out_sem,
                  *, block: int, n_steps: int, in_slots: int, out_slots: int):
    def dma_in(slot, step):
        pltpu.make_async_copy(
            x_hbm.at[pl.ds(step * block, block), :],
            x_buf.at[slot], in_sem.at[slot]).start()

    def wait_in(slot):
        pltpu.make_async_copy(
            x_hbm.at[pl.ds(0, block), :],
            x_buf.at[slot], in_sem.at[slot]).wait()

    def dma_out(slot, step):
        pltpu.make_async_copy(
            o_buf.at[slot],
            o_hbm.at[pl.ds(step * block, block), :], out_sem.at[slot]).start()

    def wait_out(slot):
        pltpu.make_async_copy(
            o_buf.at[slot],
            o_hbm.at[pl.ds(0, block), :], out_sem.at[slot]).wait()

    for s in range(min(in_slots - 1, n_steps)):
        dma_in(s, s)
    w = w_ref[...].astype(jnp.bfloat16)
    g_buf[...] = jax.lax.dot_general(
        w, w, (((1,), (1,)), ((), ())),
        preferred_element_type=jnp.float32).astype(jnp.bfloat16)

    def body(step, _):
        cur_in = jax.lax.rem(step, in_slots)
        cur_out = jax.lax.rem(step, out_slots)

        @pl.when(step + in_slots - 1 < n_steps)
        def _():
            dma_in(jax.lax.rem(step + in_slots - 1, in_slots),
                   step + in_slots - 1)

        wait_in(cur_in)

        @pl.when(step >= out_slots)
        def _():
            wait_out(cur_out)

        xt = x_buf[cur_in].astype(jnp.bfloat16)
        y = jnp.dot(xt, g_buf[...], preferred_element_type=jnp.float32)
        o_buf[cur_out] = jnp.maximum(y + b_ref[...], 0.0).astype(o_buf.dtype)

        dma_out(cur_out, step)
        return ()

    jax.lax.fori_loop(0, n_steps, body, (), unroll=False)
    for s in range(max(n_steps - out_slots, 0), n_steps):
        wait_out(s % out_slots)


def kernel(x, w1, b):
    n_data, n_feat = x.shape
    nf_w, n_hidden = w1.shape
    assert nf_w == n_feat

    b2d = b.reshape(1, n_feat)

    block = n_data
    for cand in (1024, 512, 256, 128, 64, 32, 16, 8):
        if n_data % cand == 0:
            block = cand
            break
    n_steps = n_data // block
    in_slots = min(5, max(n_steps, 2))
    out_slots = 3

    cost = pl.CostEstimate(
        flops=2 * n_data * n_feat * n_feat + 2 * n_feat * n_feat * n_hidden,
        transcendentals=0,
        bytes_accessed=2 * n_data * n_feat * 4 + n_feat * n_hidden * 4,
    )
    return pl.pallas_call(
        functools.partial(_fused_kernel, block=block, n_steps=n_steps,
                          in_slots=in_slots, out_slots=out_slots),
        out_shape=jax.ShapeDtypeStruct((n_data, n_feat), x.dtype),
        in_specs=[
            pl.BlockSpec(memory_space=pltpu.MemorySpace.HBM),
            pl.BlockSpec(memory_space=pltpu.MemorySpace.VMEM),
            pl.BlockSpec(memory_space=pltpu.MemorySpace.VMEM),
        ],
        out_specs=pl.BlockSpec(memory_space=pltpu.MemorySpace.HBM),
        scratch_shapes=[
            pltpu.VMEM((in_slots, block, n_feat), x.dtype),
            pltpu.VMEM((out_slots, block, n_feat), x.dtype),
            pltpu.VMEM((n_feat, n_feat), jnp.bfloat16),
            pltpu.SemaphoreType.DMA((in_slots,)),
            pltpu.SemaphoreType.DMA((out_slots,)),
        ],
        cost_estimate=cost,
        compiler_params=pltpu.CompilerParams(
            vmem_limit_bytes=58 * 1024 * 1024,
        ),
    )(x, w1, b2d)
